# baseline (speedup 1.0000x reference)
import functools

import jax
import jax.numpy as jnp
from jax import lax
from jax.experimental import pallas as pl
from jax.experimental.pallas import tpu as pltpu
from jax.experimental.pallas import tpu_sc as plsc

_NUM_CORES = 2
_NUM_SUBCORES = 16
_NW = _NUM_CORES * _NUM_SUBCORES
_CHUNK = 128


def _sc_gather_wide(wide, qc, batch, wdim):
    b_per_w = batch // _NW
    n_chunks = b_per_w // _CHUNK

    mesh = plsc.VectorSubcoreMesh(core_axis_name="c", subcore_axis_name="s")

    @functools.partial(
        pl.kernel,
        out_type=jax.ShapeDtypeStruct((batch, wdim), jnp.float32),
        mesh=mesh,
        scratch_types=[
            pltpu.VMEM((n_chunks, _CHUNK), jnp.int32),
            pltpu.VMEM((b_per_w, wdim), jnp.float32),
            pltpu.SemaphoreType.DMA,
        ],
    )
    def gather_kernel(wide_hbm, q_hbm, out_hbm, idx_v, rows_v, sem):
        wid = lax.axis_index("s") * _NUM_CORES + lax.axis_index("c")
        base = wid * b_per_w
        pltpu.sync_copy(q_hbm.at[wid], idx_v)
        copies = [
            pltpu.async_copy(
                wide_hbm.at[idx_v.at[j]],
                rows_v.at[pl.ds(j * _CHUNK, _CHUNK)],
                sem,
            )
            for j in range(n_chunks)
        ]
        for c in copies:
            c.wait()
        pltpu.sync_copy(rows_v, out_hbm.at[pl.ds(base, b_per_w)])

    return gather_kernel(wide, qc)


def _tc_film(xT, enc2, par8, W, b2d, blk):
    dim, batch = xT.shape

    def film_kernel(enc_ref, par_ref, xT_ref, w_ref, b_ref, outT_ref):
        e2 = enc_ref[...]
        p = par_ref[...][:, :1]
        enc = jnp.where(p > 0.5, e2[:, dim:], e2[:, :dim])
        gb = (
            jnp.dot(
                enc,
                w_ref[...],
                preferred_element_type=jnp.float32,
                precision=lax.Precision.HIGHEST,
            )
            + b_ref[...]
        )
        gbT = gb.T
        outT_ref[...] = xT_ref[...] * (1.0 + gbT[:dim, :]) + gbT[dim:, :]

    return pl.pallas_call(
        film_kernel,
        grid=(batch // blk,),
        in_specs=[
            pl.BlockSpec((blk, 2 * dim), lambda i: (i, 0)),
            pl.BlockSpec((blk, 8), lambda i: (i, 0)),
            pl.BlockSpec((dim, blk), lambda i: (0, i)),
            pl.BlockSpec(W.shape, lambda i: (0, 0)),
            pl.BlockSpec(b2d.shape, lambda i: (0, 0)),
        ],
        out_specs=pl.BlockSpec((dim, blk), lambda i: (0, i)),
        out_shape=jax.ShapeDtypeStruct((dim, batch), jnp.float32),
    )(enc2, par8, xT, W, b2d)


@jax.jit
def kernel(x, label, emb_table, W, b):
    batch, dim = x.shape
    rows = emb_table.shape[0]
    idx = label.astype(jnp.int32)
    wide = emb_table.reshape(rows // 2, 2 * dim)
    qc = (idx >> 1).reshape(_NW, batch // (_NW * _CHUNK), _CHUNK)
    par8 = jnp.broadcast_to(
        (idx & 1).astype(jnp.float32).reshape(batch, 1), (batch, 8))
    enc2 = _sc_gather_wide(wide, qc, batch, 2 * dim)
    outT = _tc_film(x.T, enc2, par8, W, b.reshape(1, -1), blk=4096)
    return outT.T


# speedup vs baseline: 2.5678x; 2.5678x over previous
import functools

import jax
import jax.numpy as jnp
from jax import lax
from jax.experimental import pallas as pl
from jax.experimental.pallas import tpu as pltpu
from jax.experimental.pallas import tpu_sc as plsc

_NUM_CORES = 2
_NUM_SUBCORES = 16
_NW = _NUM_CORES * _NUM_SUBCORES
_SUBLANES = 8
_G = 16


def _sc_gather(table3, q2, s2, batch, dim):
    b_per_w = batch // _NW
    n_groups = b_per_w // _G

    mesh = plsc.VectorSubcoreMesh(core_axis_name="c", subcore_axis_name="s")

    @functools.partial(
        pl.kernel,
        out_type=jax.ShapeDtypeStruct((batch, dim), jnp.float32),
        mesh=mesh,
        scratch_types=[
            pltpu.VMEM((b_per_w,), jnp.int32),
            pltpu.VMEM((b_per_w,), jnp.int32),
            pltpu.VMEM((b_per_w, dim), jnp.float32),
            pltpu.SemaphoreType.DMA,
        ],
    )
    def gather_kernel(table_hbm, q_hbm, s_hbm, out_hbm, q_v, s_v, rows_v, sem):
        wid = lax.axis_index("s") * _NUM_CORES + lax.axis_index("c")
        base = wid * b_per_w
        pltpu.sync_copy(q_hbm.at[wid], q_v)
        pltpu.sync_copy(s_hbm.at[wid], s_v)

        def body(g, _):
            qv = q_v[pl.ds(g * _G, _G)]
            sv = s_v[pl.ds(g * _G, _G)]
            for l in range(_G):
                pltpu.async_copy(
                    table_hbm.at[qv[l], sv[l]],
                    rows_v.at[g * _G + l],
                    sem,
                )
            return _

        lax.fori_loop(0, n_groups, body, None)
        pltpu.make_async_copy(out_hbm.at[pl.ds(base, b_per_w)], rows_v, sem).wait()
        pltpu.sync_copy(rows_v, out_hbm.at[pl.ds(base, b_per_w)])

    return gather_kernel(table3, q2, s2)


def _tc_film(xT, enc, W, b2d, blk):
    dim, batch = xT.shape

    def film_kernel(enc_ref, xT_ref, w_ref, b_ref, outT_ref):
        gb = (
            jnp.dot(
                enc_ref[...],
                w_ref[...],
                preferred_element_type=jnp.float32,
                precision=lax.Precision.HIGHEST,
            )
            + b_ref[...]
        )
        gbT = gb.T
        outT_ref[...] = xT_ref[...] * (1.0 + gbT[:dim, :]) + gbT[dim:, :]

    return pl.pallas_call(
        film_kernel,
        grid=(batch // blk,),
        in_specs=[
            pl.BlockSpec((blk, dim), lambda i: (i, 0)),
            pl.BlockSpec((dim, blk), lambda i: (0, i)),
            pl.BlockSpec(W.shape, lambda i: (0, 0)),
            pl.BlockSpec(b2d.shape, lambda i: (0, 0)),
        ],
        out_specs=pl.BlockSpec((dim, blk), lambda i: (0, i)),
        out_shape=jax.ShapeDtypeStruct((dim, batch), jnp.float32),
    )(enc, xT, W, b2d)


@jax.jit
def kernel(x, label, emb_table, W, b):
    batch, dim = x.shape
    rows = emb_table.shape[0]
    idx = label.astype(jnp.int32)
    q2 = (idx // _SUBLANES).reshape(_NW, batch // _NW)
    s2 = (idx % _SUBLANES).reshape(_NW, batch // _NW)
    table3 = emb_table.reshape(rows // _SUBLANES, _SUBLANES, dim)
    enc = _sc_gather(table3, q2, s2, batch, dim)
    outT = _tc_film(x.T, enc, W, b.reshape(1, -1), blk=4096)
    return outT.T
